# Initial kernel scaffold; baseline (speedup 1.0000x reference)
import functools

import jax
import jax.numpy as jnp
from jax import lax
from jax.experimental import pallas as pl
from jax.experimental.pallas import tpu as pltpu
from jax.experimental.pallas import tpu_sc as plsc

NC, NS, L = 2, 16, 16
NW = NC * NS
CHUNK = 128
EPS = 1e-12


def _rsqrt_newton(a):
    ii = lax.bitcast_convert_type(a, jnp.int32)
    ii = jnp.int32(0x5F3759DF) - (ii >> 1)
    y = lax.bitcast_convert_type(ii, jnp.float32)
    for _ in range(3):
        y = y * (1.5 - 0.5 * a * y * y)
    return y


def _make_kernel(B, S, H, V):
    N = B * S
    assert N % CHUNK == 0 and H == 128
    n_chunks = N // CHUNK
    assert n_chunks % NW == 0
    per_w = n_chunks // NW
    HV = H // L
    inv_h = 1.0 / H

    mesh = plsc.VectorSubcoreMesh(core_axis_name="c", subcore_axis_name="s")

    @functools.partial(
        pl.kernel,
        out_type=jax.ShapeDtypeStruct((N, H), jnp.float32),
        mesh=mesh,
        scratch_types=[
            pltpu.VMEM((S, H), jnp.float32),
            pltpu.VMEM((2, H), jnp.float32),
            pltpu.VMEM((1, H), jnp.float32),
            pltpu.VMEM((H,), jnp.float32),
            pltpu.VMEM((H,), jnp.float32),
            pltpu.VMEM((CHUNK,), jnp.int32),
            pltpu.VMEM((CHUNK,), jnp.int32),
            pltpu.VMEM((CHUNK, H), jnp.float32),
            pltpu.SemaphoreType.DMA,
        ],
    )
    def k(ids_hbm, tt_hbm, word_hbm, pos_hbm, type_hbm, w_hbm, b_hbm,
          out_hbm, pos_v, type_v, dt_v, w_v, b_v, idx_v, ttc_v, buf_v, sem):
        wid = lax.axis_index("s") * NC + lax.axis_index("c")

        pltpu.sync_copy(pos_hbm.at[pl.ds(0, S)], pos_v)
        pltpu.sync_copy(type_hbm, type_v)
        pltpu.sync_copy(w_hbm, w_v)
        pltpu.sync_copy(b_hbm, b_v)
        for i in range(HV):
            sl = pl.ds(i * L, L)
            dt_v[0, sl] = type_v[1, sl] - type_v[0, sl]

        def chunk_body(g, carry):
            j = wid * per_w + g
            pltpu.sync_copy(ids_hbm.at[j], idx_v)
            pltpu.sync_copy(tt_hbm.at[j], ttc_v)
            pltpu.async_copy(word_hbm.at[idx_v], buf_v, sem).wait()
            base = lax.rem(j * CHUNK, S)

            def tok_body(t, c):
                ttf = plsc.load_gather(
                    ttc_v, [jnp.full((L,), 0, jnp.int32) + t]
                ).astype(jnp.float32)
                p = base + t
                p = jnp.where(p >= S, p - S, p)
                s = jnp.zeros((L,), jnp.float32)
                s2 = jnp.zeros((L,), jnp.float32)
                xs = []
                for i in range(HV):
                    sl = pl.ds(i * L, L)
                    x = buf_v[t, sl] + pos_v[p, sl] + type_v[0, sl] \
                        + ttf * dt_v[0, sl]
                    xs.append(x)
                    s = s + x
                    s2 = s2 + x * x
                mean = jnp.sum(s) * inv_h
                var = jnp.sum(s2) * inv_h - mean * mean
                r = _rsqrt_newton(var + EPS)
                for i in range(HV):
                    sl = pl.ds(i * L, L)
                    buf_v[t, sl] = (xs[i] - mean) * r * w_v[sl] + b_v[sl]
                return c

            lax.fori_loop(0, CHUNK, tok_body, 0)
            pltpu.sync_copy(buf_v, out_hbm.at[pl.ds(j * CHUNK, CHUNK)])
            return carry

        lax.fori_loop(0, per_w, chunk_body, 0)

    return k


def kernel(input_ids, token_type_ids, word_emb, pos_emb, type_emb,
           ln_weight, ln_bias):
    B, S = input_ids.shape
    V, H = word_emb.shape
    N = B * S
    ids2 = input_ids.astype(jnp.int32).reshape(N // CHUNK, CHUNK)
    tt2 = token_type_ids.astype(jnp.int32).reshape(N // CHUNK, CHUNK)
    k = _make_kernel(B, S, H, V)
    out = k(ids2, tt2, word_emb, pos_emb, type_emb, ln_weight, ln_bias)
    return out.reshape(B, S, H)


# speedup vs baseline: 1.5714x; 1.5714x over previous
import functools

import jax
import jax.numpy as jnp
from jax import lax
from jax.experimental import pallas as pl
from jax.experimental.pallas import tpu as pltpu
from jax.experimental.pallas import tpu_sc as plsc

NC, NS, L = 2, 16, 16
NW = NC * NS
CHUNK = 128
EPS = 1e-12


def _rsqrt_newton(a):
    ii = lax.bitcast_convert_type(a, jnp.int32)
    ii = jnp.int32(0x5F3759DF) - (ii >> 1)
    y = lax.bitcast_convert_type(ii, jnp.float32)
    for _ in range(3):
        y = y * (1.5 - 0.5 * a * y * y)
    return y


def _make_kernel(B, S, H, V):
    N = B * S
    assert N % CHUNK == 0 and H == 128
    n_chunks = N // CHUNK
    assert n_chunks % NW == 0
    per_w = n_chunks // NW
    HV = H // L
    inv_h = 1.0 / H

    mesh = plsc.VectorSubcoreMesh(core_axis_name="c", subcore_axis_name="s")

    @functools.partial(
        pl.kernel,
        out_type=jax.ShapeDtypeStruct((N, H), jnp.float32),
        mesh=mesh,
        compiler_params=pltpu.CompilerParams(needs_layout_passes=False),
        scratch_types=[
            pltpu.VMEM((S, H), jnp.float32),
            pltpu.VMEM((2, H), jnp.float32),
            pltpu.VMEM((1, H), jnp.float32),
            pltpu.VMEM((H,), jnp.float32),
            pltpu.VMEM((H,), jnp.float32),
            pltpu.VMEM((CHUNK,), jnp.int32),
            pltpu.VMEM((CHUNK,), jnp.int32),
            pltpu.VMEM((CHUNK, H), jnp.float32),
            pltpu.SemaphoreType.DMA,
        ],
    )
    def k(ids_hbm, tt_hbm, word_hbm, pos_hbm, type_hbm, w_hbm, b_hbm,
          out_hbm, pos_v, type_v, dt_v, w_v, b_v, idx_v, ttc_v, buf_v, sem):
        wid = lax.axis_index("s") * NC + lax.axis_index("c")

        pltpu.sync_copy(pos_hbm.at[pl.ds(0, S)], pos_v)
        pltpu.sync_copy(type_hbm, type_v)
        pltpu.sync_copy(w_hbm, w_v)
        pltpu.sync_copy(b_hbm, b_v)
        for i in range(HV):
            sl = pl.ds(i * L, L)
            dt_v[0, sl] = type_v[1, sl] - type_v[0, sl]

        def chunk_body(g, carry):
            j = wid * per_w + g
            pltpu.sync_copy(ids_hbm.at[j], idx_v)
            pltpu.sync_copy(tt_hbm.at[j], ttc_v)
            pltpu.async_copy(word_hbm.at[idx_v], buf_v, sem).wait()
            base = lax.rem(j * CHUNK, S)

            def grp_body(gg, c):
                t0 = gg * L
                ttf16 = ttc_v[pl.ds(t0, L)].astype(jnp.float32)
                for u in range(L):
                    t = t0 + u
                    ttf = ttf16[u]
                    p = base + t
                    p = jnp.where(p >= S, p - S, p)
                    s = jnp.zeros((L,), jnp.float32)
                    s2 = jnp.zeros((L,), jnp.float32)
                    xs = []
                    for i in range(HV):
                        sl = pl.ds(i * L, L)
                        x = buf_v[t, sl] + pos_v[p, sl] + type_v[0, sl] \
                            + ttf * dt_v[0, sl]
                        xs.append(x)
                        s = s + x
                        s2 = s2 + x * x
                    mean = jnp.sum(s) * inv_h
                    var = jnp.sum(s2) * inv_h - mean * mean
                    r = _rsqrt_newton(var + EPS)
                    for i in range(HV):
                        sl = pl.ds(i * L, L)
                        buf_v[t, sl] = (xs[i] - mean) * r * w_v[sl] + b_v[sl]
                return c

            lax.fori_loop(0, CHUNK // L, grp_body, 0)
            pltpu.sync_copy(buf_v, out_hbm.at[pl.ds(j * CHUNK, CHUNK)])
            return carry

        lax.fori_loop(0, per_w, chunk_body, 0)

    return k


def kernel(input_ids, token_type_ids, word_emb, pos_emb, type_emb,
           ln_weight, ln_bias):
    B, S = input_ids.shape
    V, H = word_emb.shape
    N = B * S
    ids2 = input_ids.astype(jnp.int32).reshape(N // CHUNK, CHUNK)
    tt2 = token_type_ids.astype(jnp.int32).reshape(N // CHUNK, CHUNK)
    k = _make_kernel(B, S, H, V)
    out = k(ids2, tt2, word_emb, pos_emb, type_emb, ln_weight, ln_bias)
    return out.reshape(B, S, H)


# speedup vs baseline: 4.9160x; 3.1285x over previous
import functools

import jax
import jax.numpy as jnp
from jax import lax
from jax.experimental import pallas as pl
from jax.experimental.pallas import tpu as pltpu
from jax.experimental.pallas import tpu_sc as plsc

NC, NS, L = 2, 16, 16
NW = NC * NS
CHUNK = 128
EPS = 1e-12


def _rsqrt_newton(a):
    ii = lax.bitcast_convert_type(a, jnp.int32)
    ii = jnp.int32(0x5F3759DF) - (ii >> 1)
    y = lax.bitcast_convert_type(ii, jnp.float32)
    for _ in range(3):
        y = y * (1.5 - 0.5 * a * y * y)
    return y


def _make_kernel(B, S, H, V):
    N = B * S
    assert N % CHUNK == 0 and H == 128
    n_chunks = N // CHUNK
    assert n_chunks % NW == 0
    per_w = n_chunks // NW
    assert per_w % 2 == 0
    HV = H // L
    inv_h = 1.0 / H

    mesh = plsc.VectorSubcoreMesh(core_axis_name="c", subcore_axis_name="s")

    @functools.partial(
        pl.kernel,
        out_type=jax.ShapeDtypeStruct((N, H), jnp.float32),
        mesh=mesh,
        compiler_params=pltpu.CompilerParams(needs_layout_passes=False),
        scratch_types=[
            pltpu.VMEM((2 * S, H), jnp.float32),
            pltpu.VMEM((2, H), jnp.float32),
            pltpu.VMEM((H,), jnp.float32),
            pltpu.VMEM((H,), jnp.float32),
            pltpu.VMEM((CHUNK,), jnp.int32),
            pltpu.VMEM((CHUNK,), jnp.int32),
            pltpu.VMEM((CHUNK,), jnp.int32),
            pltpu.VMEM((CHUNK,), jnp.int32),
            pltpu.VMEM((CHUNK, H), jnp.float32),
            pltpu.VMEM((CHUNK, H), jnp.float32),
            pltpu.VMEM((CHUNK, H), jnp.float32),
            pltpu.VMEM((CHUNK, H), jnp.float32),
            pltpu.SemaphoreType.DMA,
            pltpu.SemaphoreType.DMA,
            pltpu.SemaphoreType.DMA,
            pltpu.SemaphoreType.DMA,
        ],
    )
    def k(ids_hbm, tt_hbm, word_hbm, pos_hbm, type_hbm, w_hbm, b_hbm,
          out_hbm, ptt_v, type_v, w_v, b_v, idx_a, idx_b, ttc_a, ttc_b,
          gbuf_a, gbuf_b, obuf_a, obuf_b, gsem_a, gsem_b, ssem_a, ssem_b):
        wid = lax.axis_index("s") * NC + lax.axis_index("c")
        j0 = wid * per_w

        pltpu.sync_copy(pos_hbm.at[pl.ds(0, S)], ptt_v.at[pl.ds(0, S)])
        pltpu.sync_copy(pos_hbm.at[pl.ds(0, S)], ptt_v.at[pl.ds(S, S)])
        pltpu.sync_copy(type_hbm, type_v)
        pltpu.sync_copy(w_hbm, w_v)
        pltpu.sync_copy(b_hbm, b_v)
        t0s = [type_v[0, pl.ds(i * L, L)] for i in range(HV)]
        t1s = [type_v[1, pl.ds(i * L, L)] for i in range(HV)]

        def build_body(p, c):
            for i in range(HV):
                sl = pl.ds(i * L, L)
                ptt_v[p, sl] = ptt_v[p, sl] + t0s[i]
                ptt_v[S + p, sl] = ptt_v[S + p, sl] + t1s[i]
            return c

        lax.fori_loop(0, S, build_body, 0)

        ws = [w_v[pl.ds(i * L, L)] for i in range(HV)]
        bs = [b_v[pl.ds(i * L, L)] for i in range(HV)]

        def compute_chunk(j, gbuf, obuf, ttc):
            base = lax.rem(j * CHUNK, S)

            def grp_body(gg, c):
                t0 = gg * L
                tt16 = ttc[pl.ds(t0, L)]
                for u in range(L):
                    t = t0 + u
                    tt = tt16[u]
                    p = base + t
                    p = jnp.where(p >= S, p - S, p)
                    p2 = tt * S + p
                    s = jnp.zeros((L,), jnp.float32)
                    s2 = jnp.zeros((L,), jnp.float32)
                    xs = []
                    for i in range(HV):
                        sl = pl.ds(i * L, L)
                        x = gbuf[t, sl] + ptt_v[p2, sl]
                        xs.append(x)
                        s = s + x
                        s2 = s2 + x * x
                    mean = jnp.sum(s) * inv_h
                    var = jnp.sum(s2) * inv_h - mean * mean
                    r = _rsqrt_newton(var + EPS)
                    for i in range(HV):
                        sl = pl.ds(i * L, L)
                        obuf[t, sl] = (xs[i] - mean) * r * ws[i] + bs[i]
                return c

            lax.fori_loop(0, CHUNK // L, grp_body, 0)

        pltpu.sync_copy(ids_hbm.at[j0], idx_a)
        pltpu.sync_copy(tt_hbm.at[j0], ttc_a)
        pltpu.async_copy(word_hbm.at[idx_a], gbuf_a, gsem_a)

        def half(k_, jj, jn, idx_t, ttc_t, idx_n, gbuf_t, gbuf_n, obuf_t,
                 gsem_t, gsem_n, ssem_t):
            pltpu.sync_copy(ids_hbm.at[jn], idx_n)
            pltpu.async_copy(word_hbm.at[idx_n], gbuf_n, gsem_n)
            pltpu.make_async_copy(word_hbm.at[idx_t], gbuf_t, gsem_t).wait()

            @pl.when(k_ > 0)
            def _():
                pltpu.make_async_copy(
                    obuf_t, out_hbm.at[pl.ds(jj * CHUNK, CHUNK)], ssem_t
                ).wait()

            compute_chunk(jj, gbuf_t, obuf_t, ttc_t)
            pltpu.async_copy(
                obuf_t, out_hbm.at[pl.ds(jj * CHUNK, CHUNK)], ssem_t)

        def pair_body(k_, c):
            ja = j0 + 2 * k_
            jb = ja + 1
            jn = j0 + jnp.minimum(2 * k_ + 2, per_w - 1)
            pltpu.sync_copy(tt_hbm.at[jb], ttc_b)
            half(k_, ja, jb, idx_a, ttc_a, idx_b, gbuf_a, gbuf_b,
                 obuf_a, gsem_a, gsem_b, ssem_a)
            pltpu.sync_copy(tt_hbm.at[jn], ttc_a)
            half(k_, jb, jn, idx_b, ttc_b, idx_a, gbuf_b, gbuf_a,
                 obuf_b, gsem_b, gsem_a, ssem_b)
            return c

        lax.fori_loop(0, per_w // 2, pair_body, 0)

        pltpu.make_async_copy(word_hbm.at[idx_a], gbuf_a, gsem_a).wait()
        pltpu.make_async_copy(
            obuf_a, out_hbm.at[pl.ds(j0 * CHUNK, CHUNK)], ssem_a).wait()
        pltpu.make_async_copy(
            obuf_b, out_hbm.at[pl.ds(j0 * CHUNK, CHUNK)], ssem_b).wait()

    return k


def kernel(input_ids, token_type_ids, word_emb, pos_emb, type_emb,
           ln_weight, ln_bias):
    B, S = input_ids.shape
    V, H = word_emb.shape
    N = B * S
    ids2 = input_ids.astype(jnp.int32).reshape(N // CHUNK, CHUNK)
    tt2 = token_type_ids.astype(jnp.int32).reshape(N // CHUNK, CHUNK)
    k = _make_kernel(B, S, H, V)
    out = k(ids2, tt2, word_emb, pos_emb, type_emb, ln_weight, ln_bias)
    return out.reshape(B, S, H)


# speedup vs baseline: 9.1941x; 1.8703x over previous
import functools

import jax
import jax.numpy as jnp
from jax import lax
from jax.experimental import pallas as pl
from jax.experimental.pallas import tpu as pltpu
from jax.experimental.pallas import tpu_sc as plsc

NC, NS, L = 2, 16, 16
NW = NC * NS
CHUNK = 128
EPS = 1e-12


def _rsqrt_newton(a):
    ii = lax.bitcast_convert_type(a, jnp.int32)
    ii = jnp.full(ii.shape, 0x5F3759DF, jnp.int32) - (ii >> 1)
    y = lax.bitcast_convert_type(ii, jnp.float32)
    h = -0.5 * a
    for _ in range(2):
        y = y * (1.5 + h * y * y)
    return y


def _gather16(v, idx):
    return v.at[idx].get(mode="promise_in_bounds")


def _make_kernel(B, S, H, V):
    N = B * S
    assert N % CHUNK == 0 and H == 128
    n_chunks = N // CHUNK
    assert n_chunks % NW == 0
    per_w = n_chunks // NW
    assert per_w % 2 == 0
    HV = H // L
    inv_h = 1.0 / H

    mesh = plsc.VectorSubcoreMesh(core_axis_name="c", subcore_axis_name="s")

    @functools.partial(
        pl.kernel,
        out_type=jax.ShapeDtypeStruct((N, H), jnp.float32),
        mesh=mesh,
        compiler_params=pltpu.CompilerParams(needs_layout_passes=False),
        scratch_types=[
            pltpu.VMEM((S, H), jnp.float32),
            pltpu.VMEM((2, H), jnp.float32),
            pltpu.VMEM((H,), jnp.float32),
            pltpu.VMEM((H,), jnp.float32),
            pltpu.VMEM((CHUNK,), jnp.int32),
            pltpu.VMEM((CHUNK,), jnp.int32),
            pltpu.VMEM((CHUNK,), jnp.int32),
            pltpu.VMEM((CHUNK,), jnp.int32),
            pltpu.VMEM((CHUNK, H), jnp.float32),
            pltpu.VMEM((CHUNK, H), jnp.float32),
            pltpu.VMEM((CHUNK, H), jnp.float32),
            pltpu.VMEM((CHUNK, H), jnp.float32),
            pltpu.SemaphoreType.DMA,
            pltpu.SemaphoreType.DMA,
            pltpu.SemaphoreType.DMA,
            pltpu.SemaphoreType.DMA,
        ],
    )
    def k(ids_hbm, tt_hbm, word_hbm, pos_hbm, type_hbm, w_hbm, b_hbm,
          out_hbm, ptt_v, type_v, w_v, b_v, idx_a, idx_b, ttc_a, ttc_b,
          gbuf_a, gbuf_b, obuf_a, obuf_b, gsem_a, gsem_b, ssem_a, ssem_b):
        wid = lax.axis_index("s") * NC + lax.axis_index("c")
        j0 = wid * per_w

        pltpu.sync_copy(pos_hbm.at[pl.ds(0, S)], ptt_v)
        pltpu.sync_copy(type_hbm, type_v)
        pltpu.sync_copy(w_hbm, w_v)
        pltpu.sync_copy(b_hbm, b_v)
        t0s = [type_v[0, pl.ds(i * L, L)] for i in range(HV)]

        def build_body(p, c):
            for i in range(HV):
                sl = pl.ds(i * L, L)
                ptt_v[p, sl] = ptt_v[p, sl] + t0s[i]
            return c

        lax.fori_loop(0, S, build_body, 0)

        ws = [w_v[pl.ds(i * L, L)] for i in range(HV)]
        bs = [b_v[pl.ds(i * L, L)] for i in range(HV)]
        dts = [type_v[1, pl.ds(i * L, L)] - type_v[0, pl.ds(i * L, L)]
               for i in range(HV)]

        iota = lax.iota(jnp.int32, L)
        lane_idx = [iota * 0 + u for u in range(L)]
        rot_idx = [(iota + sh) & (L - 1) for sh in (8, 4, 2, 1)]

        def lane_sum(v):
            for ridx in rot_idx:
                v = v + _gather16(v, ridx)
            return v

        def compute_chunk(j, gbuf, obuf, ttc):
            base = lax.rem(j * CHUNK, S)

            def grp_body(gg, c):
                t0 = gg * L
                ttf16 = ttc[pl.ds(t0, L)].astype(jnp.float32)
                for u in range(L):
                    t = t0 + u
                    ttb = _gather16(ttf16, lane_idx[u])
                    p = base + t
                    p = jnp.where(p >= S, p - S, p)
                    xs = []
                    for i in range(HV):
                        sl = pl.ds(i * L, L)
                        x = (gbuf[t, sl] + ptt_v[p, sl]) + ttb * dts[i]
                        xs.append(x)
                    vs = list(xs)
                    qs = [x * x for x in xs]
                    while len(vs) > 1:
                        vs = [vs[i_] + vs[i_ + 1]
                              for i_ in range(0, len(vs), 2)]
                        qs = [qs[i_] + qs[i_ + 1]
                              for i_ in range(0, len(qs), 2)]
                    mean = lane_sum(vs[0]) * inv_h
                    var = lane_sum(qs[0]) * inv_h - mean * mean
                    r = _rsqrt_newton(var + EPS)
                    for i in range(HV):
                        sl = pl.ds(i * L, L)
                        obuf[t, sl] = (xs[i] - mean) * r * ws[i] + bs[i]
                return c

            lax.fori_loop(0, CHUNK // L, grp_body, 0)

        pltpu.sync_copy(ids_hbm.at[j0], idx_a)
        pltpu.sync_copy(tt_hbm.at[j0], ttc_a)
        pltpu.async_copy(word_hbm.at[idx_a], gbuf_a, gsem_a)

        def half(k_, jj, jn, idx_t, ttc_t, idx_n, gbuf_t, gbuf_n, obuf_t,
                 gsem_t, gsem_n, ssem_t):
            pltpu.sync_copy(ids_hbm.at[jn], idx_n)
            pltpu.async_copy(word_hbm.at[idx_n], gbuf_n, gsem_n)
            pltpu.make_async_copy(word_hbm.at[idx_t], gbuf_t, gsem_t).wait()

            @pl.when(k_ > 0)
            def _():
                pltpu.make_async_copy(
                    obuf_t, out_hbm.at[pl.ds(jj * CHUNK, CHUNK)], ssem_t
                ).wait()

            compute_chunk(jj, gbuf_t, obuf_t, ttc_t)
            pltpu.async_copy(
                obuf_t, out_hbm.at[pl.ds(jj * CHUNK, CHUNK)], ssem_t)

        def pair_body(k_, c):
            ja = j0 + 2 * k_
            jb = ja + 1
            jn = j0 + jnp.minimum(2 * k_ + 2, per_w - 1)
            pltpu.sync_copy(tt_hbm.at[jb], ttc_b)
            half(k_, ja, jb, idx_a, ttc_a, idx_b, gbuf_a, gbuf_b,
                 obuf_a, gsem_a, gsem_b, ssem_a)
            pltpu.sync_copy(tt_hbm.at[jn], ttc_a)
            half(k_, jb, jn, idx_b, ttc_b, idx_a, gbuf_b, gbuf_a,
                 obuf_b, gsem_b, gsem_a, ssem_b)
            return c

        lax.fori_loop(0, per_w // 2, pair_body, 0)

        pltpu.make_async_copy(word_hbm.at[idx_a], gbuf_a, gsem_a).wait()
        pltpu.make_async_copy(
            obuf_a, out_hbm.at[pl.ds(j0 * CHUNK, CHUNK)], ssem_a).wait()
        pltpu.make_async_copy(
            obuf_b, out_hbm.at[pl.ds(j0 * CHUNK, CHUNK)], ssem_b).wait()

    return k


def kernel(input_ids, token_type_ids, word_emb, pos_emb, type_emb,
           ln_weight, ln_bias):
    B, S = input_ids.shape
    V, H = word_emb.shape
    N = B * S
    ids2 = input_ids.astype(jnp.int32).reshape(N // CHUNK, CHUNK)
    tt2 = token_type_ids.astype(jnp.int32).reshape(N // CHUNK, CHUNK)
    k = _make_kernel(B, S, H, V)
    out = k(ids2, tt2, word_emb, pos_emb, type_emb, ln_weight, ln_bias)
    return out.reshape(B, S, H)


# speedup vs baseline: 9.2085x; 1.0016x over previous
import functools

import jax
import jax.numpy as jnp
from jax import lax
from jax.experimental import pallas as pl
from jax.experimental.pallas import tpu as pltpu
from jax.experimental.pallas import tpu_sc as plsc

NC, NS, L = 2, 16, 16
NW = NC * NS
CHUNK = 128
EPS = 1e-12


def _rsqrt_newton(a):
    ii = lax.bitcast_convert_type(a, jnp.int32)
    ii = jnp.full(ii.shape, 0x5F3759DF, jnp.int32) - (ii >> 1)
    y = lax.bitcast_convert_type(ii, jnp.float32)
    h = -0.5 * a
    for _ in range(2):
        y = y * (1.5 + h * y * y)
    return y


def _gather16(v, idx):
    return v.at[idx].get(mode="promise_in_bounds")


def _make_sc_kernel(B, S, H, V):
    N = B * S
    assert N % CHUNK == 0 and H == 128
    n_chunks = N // CHUNK
    assert n_chunks % NW == 0
    per_w = n_chunks // NW
    assert per_w % 2 == 0
    HV = H // L
    GRP = CHUNK // L
    inv_h = 1.0 / H

    mesh = plsc.VectorSubcoreMesh(core_axis_name="c", subcore_axis_name="s")

    @functools.partial(
        pl.kernel,
        out_type=jax.ShapeDtypeStruct((N, H), jnp.float32),
        mesh=mesh,
        compiler_params=pltpu.CompilerParams(needs_layout_passes=False),
        scratch_types=[
            pltpu.VMEM((H,), jnp.float32),
            pltpu.VMEM((H,), jnp.float32),
            pltpu.VMEM((CHUNK,), jnp.int32),
            pltpu.VMEM((CHUNK,), jnp.int32),
            pltpu.VMEM((CHUNK,), jnp.int32),
            pltpu.VMEM((CHUNK,), jnp.int32),
            pltpu.VMEM((CHUNK,), jnp.int32),
            pltpu.VMEM((CHUNK,), jnp.int32),
            pltpu.VMEM((CHUNK, H), jnp.float32),
            pltpu.VMEM((CHUNK, H), jnp.float32),
            pltpu.VMEM((CHUNK, H), jnp.float32),
            pltpu.VMEM((CHUNK, H), jnp.float32),
            pltpu.VMEM((CHUNK, H), jnp.float32),
            pltpu.VMEM((CHUNK, H), jnp.float32),
            pltpu.SemaphoreType.DMA,
            pltpu.SemaphoreType.DMA,
            pltpu.SemaphoreType.DMA,
            pltpu.SemaphoreType.DMA,
            pltpu.SemaphoreType.DMA,
            pltpu.SemaphoreType.DMA,
        ],
    )
    def k(ids_hbm, tt_hbm, word_hbm, ptt_hbm, w_hbm, b_hbm,
          out_hbm, w_v, b_v, idx_a, idx_b, ttc_a, ttc_b, pix_a, pix_b,
          gbuf_a, gbuf_b, pbuf_a, pbuf_b, obuf_a, obuf_b,
          gsem_a, gsem_b, psem_a, psem_b, ssem_a, ssem_b):
        wid = lax.axis_index("s") * NC + lax.axis_index("c")
        j0 = wid * per_w

        pltpu.sync_copy(w_hbm, w_v)
        pltpu.sync_copy(b_hbm, b_v)
        ws = [w_v[pl.ds(i * L, L)] for i in range(HV)]
        bs = [b_v[pl.ds(i * L, L)] for i in range(HV)]

        iota = lax.iota(jnp.int32, L)
        lane_idx = [iota * 0 + u for u in range(L)]
        rot_idx = [(iota + sh) & (L - 1) for sh in (8, 4, 2, 1)]

        def lane_sum(v):
            for ridx in rot_idx:
                v = v + _gather16(v, ridx)
            return v

        def stage_chunk(j, idx_t, ttc_t, pix_t):
            pltpu.sync_copy(ids_hbm.at[j], idx_t)
            pltpu.sync_copy(tt_hbm.at[j], ttc_t)
            base = lax.rem(j * CHUNK, S)
            for g in range(GRP):
                sl = pl.ds(g * L, L)
                pv = base + g * L + iota
                pv = jnp.where(pv >= S, pv - S, pv)
                pix_t[sl] = ttc_t[sl] * S + pv

        def start_gathers(idx_t, pix_t, gbuf_t, pbuf_t, gsem_t, psem_t):
            pltpu.async_copy(word_hbm.at[idx_t], gbuf_t, gsem_t)
            pltpu.async_copy(ptt_hbm.at[pix_t], pbuf_t, psem_t)

        def compute_chunk(j, gbuf, pbuf, obuf):
            def grp_body(gg, c):
                t0 = gg * L
                acc_s = jnp.zeros((L,), jnp.float32)
                acc_q = jnp.zeros((L,), jnp.float32)
                for u in range(L):
                    t = t0 + u
                    xs = []
                    for i in range(HV):
                        sl = pl.ds(i * L, L)
                        x = gbuf[t, sl] + pbuf[t, sl]
                        obuf[t, sl] = x
                        xs.append(x)
                    vs = list(xs)
                    qs = [x * x for x in xs]
                    while len(vs) > 1:
                        vs = [vs[i_] + vs[i_ + 1]
                              for i_ in range(0, len(vs), 2)]
                        qs = [qs[i_] + qs[i_ + 1]
                              for i_ in range(0, len(qs), 2)]
                    msk = iota == u
                    acc_s = jnp.where(msk, lane_sum(vs[0]), acc_s)
                    acc_q = jnp.where(msk, lane_sum(qs[0]), acc_q)
                mean16 = acc_s * inv_h
                var16 = acc_q * inv_h - mean16 * mean16
                r16 = _rsqrt_newton(var16 + EPS)
                for u in range(L):
                    t = t0 + u
                    mb = _gather16(mean16, lane_idx[u])
                    rb = _gather16(r16, lane_idx[u])
                    for i in range(HV):
                        sl = pl.ds(i * L, L)
                        obuf[t, sl] = (obuf[t, sl] - mb) * rb * ws[i] + bs[i]
                return c

            lax.fori_loop(0, GRP, grp_body, 0)

        stage_chunk(j0, idx_a, ttc_a, pix_a)
        start_gathers(idx_a, pix_a, gbuf_a, pbuf_a, gsem_a, psem_a)

        def half(k_, jj, jn, idx_t, pix_t, ttc_n, idx_n, pix_n,
                 gbuf_t, pbuf_t, gbuf_n, pbuf_n, obuf_t,
                 gsem_t, psem_t, gsem_n, psem_n, ssem_t):
            stage_chunk(jn, idx_n, ttc_n, pix_n)
            start_gathers(idx_n, pix_n, gbuf_n, pbuf_n, gsem_n, psem_n)
            pltpu.make_async_copy(word_hbm.at[idx_t], gbuf_t, gsem_t).wait()
            pltpu.make_async_copy(ptt_hbm.at[pix_t], pbuf_t, psem_t).wait()

            @pl.when(k_ > 0)
            def _():
                pltpu.make_async_copy(
                    obuf_t, out_hbm.at[pl.ds(jj * CHUNK, CHUNK)], ssem_t
                ).wait()

            compute_chunk(jj, gbuf_t, pbuf_t, obuf_t)
            pltpu.async_copy(
                obuf_t, out_hbm.at[pl.ds(jj * CHUNK, CHUNK)], ssem_t)

        def pair_body(k_, c):
            ja = j0 + 2 * k_
            jb = ja + 1
            jn = j0 + jnp.minimum(2 * k_ + 2, per_w - 1)
            half(k_, ja, jb, idx_a, pix_a, ttc_b, idx_b, pix_b,
                 gbuf_a, pbuf_a, gbuf_b, pbuf_b, obuf_a,
                 gsem_a, psem_a, gsem_b, psem_b, ssem_a)
            half(k_, jb, jn, idx_b, pix_b, ttc_a, idx_a, pix_a,
                 gbuf_b, pbuf_b, gbuf_a, pbuf_a, obuf_b,
                 gsem_b, psem_b, gsem_a, psem_a, ssem_b)
            return c

        lax.fori_loop(0, per_w // 2, pair_body, 0)

        pltpu.make_async_copy(word_hbm.at[idx_a], gbuf_a, gsem_a).wait()
        pltpu.make_async_copy(ptt_hbm.at[pix_a], pbuf_a, psem_a).wait()
        pltpu.make_async_copy(
            obuf_a, out_hbm.at[pl.ds(j0 * CHUNK, CHUNK)], ssem_a).wait()
        pltpu.make_async_copy(
            obuf_b, out_hbm.at[pl.ds(j0 * CHUNK, CHUNK)], ssem_b).wait()

    return k


def _build_ptt(pos_emb, type_emb, S, H):
    def body(pos_ref, type_ref, out_ref):
        p = pos_ref[pl.ds(0, S), :]
        out_ref[pl.ds(0, S), :] = p + type_ref[0:1, :]
        out_ref[pl.ds(S, S), :] = p + type_ref[1:2, :]

    return pl.pallas_call(
        body,
        out_shape=jax.ShapeDtypeStruct((2 * S, H), jnp.float32),
    )(pos_emb, type_emb)


def kernel(input_ids, token_type_ids, word_emb, pos_emb, type_emb,
           ln_weight, ln_bias):
    B, S = input_ids.shape
    V, H = word_emb.shape
    N = B * S
    ids2 = input_ids.astype(jnp.int32).reshape(N // CHUNK, CHUNK)
    tt2 = token_type_ids.astype(jnp.int32).reshape(N // CHUNK, CHUNK)
    ptt = _build_ptt(pos_emb, type_emb, S, H)
    k = _make_sc_kernel(B, S, H, V)
    out = k(ids2, tt2, word_emb, ptt, ln_weight, ln_bias)
    return out.reshape(B, S, H)


# speedup vs baseline: 10.8096x; 1.1739x over previous
import functools

import jax
import jax.numpy as jnp
from jax import lax
from jax.experimental import pallas as pl
from jax.experimental.pallas import tpu as pltpu
from jax.experimental.pallas import tpu_sc as plsc

NC, NS, L = 2, 16, 16
NW = NC * NS
CHUNK = 128
EPS = 1e-12


def _rsqrt_newton(a):
    ii = lax.bitcast_convert_type(a, jnp.int32)
    ii = jnp.full(ii.shape, 0x5F3759DF, jnp.int32) - (ii >> 1)
    y = lax.bitcast_convert_type(ii, jnp.float32)
    h = -0.5 * a
    for _ in range(2):
        y = y * (1.5 + h * y * y)
    return y


def _gather16(v, idx):
    return v.at[idx].get(mode="promise_in_bounds")


def _make_sc_kernel(B, S, H, V):
    N = B * S
    assert N % CHUNK == 0 and H == 128
    n_chunks = N // CHUNK
    assert n_chunks % NW == 0
    per_w = n_chunks // NW
    assert per_w % 2 == 0
    HV = H // L
    GRP = CHUNK // L
    inv_h = 1.0 / H

    mesh = plsc.VectorSubcoreMesh(core_axis_name="c", subcore_axis_name="s")

    @functools.partial(
        pl.kernel,
        out_type=jax.ShapeDtypeStruct((N, H), jnp.float32),
        mesh=mesh,
        compiler_params=pltpu.CompilerParams(needs_layout_passes=False),
        scratch_types=[
            pltpu.VMEM((H,), jnp.float32),
            pltpu.VMEM((H,), jnp.float32),
            pltpu.VMEM((per_w * CHUNK,), jnp.int32),
            pltpu.VMEM((per_w * CHUNK,), jnp.int32),
            pltpu.VMEM((per_w * CHUNK,), jnp.int32),
            pltpu.VMEM((CHUNK, H), jnp.float32),
            pltpu.VMEM((CHUNK, H), jnp.float32),
            pltpu.VMEM((CHUNK, H), jnp.float32),
            pltpu.VMEM((CHUNK, H), jnp.float32),
            pltpu.VMEM((CHUNK, H), jnp.float32),
            pltpu.VMEM((CHUNK, H), jnp.float32),
            pltpu.SemaphoreType.DMA,
            pltpu.SemaphoreType.DMA,
            pltpu.SemaphoreType.DMA,
            pltpu.SemaphoreType.DMA,
            pltpu.SemaphoreType.DMA,
            pltpu.SemaphoreType.DMA,
        ],
    )
    def k(ids_hbm, tt_hbm, word_hbm, ptt_hbm, w_hbm, b_hbm,
          out_hbm, w_v, b_v, ids_v, ttc_v, pix_v,
          gbuf_a, gbuf_b, pbuf_a, pbuf_b, obuf_a, obuf_b,
          gsem_a, gsem_b, psem_a, psem_b, ssem_a, ssem_b):
        wid = lax.axis_index("s") * NC + lax.axis_index("c")
        j0 = wid * per_w

        pltpu.sync_copy(w_hbm, w_v)
        pltpu.sync_copy(b_hbm, b_v)
        pltpu.sync_copy(ids_hbm.at[pl.ds(j0 * CHUNK, per_w * CHUNK)], ids_v)
        pltpu.sync_copy(tt_hbm.at[pl.ds(j0 * CHUNK, per_w * CHUNK)], ttc_v)

        ws = [w_v[pl.ds(i * L, L)] for i in range(HV)]
        bs = [b_v[pl.ds(i * L, L)] for i in range(HV)]

        iota = lax.iota(jnp.int32, L)
        z16 = iota * 0
        rot_idx = [(iota + sh) & (L - 1) for sh in (8, 4, 2, 1)]

        def lane_sum(v):
            for ridx in rot_idx:
                v = v + _gather16(v, ridx)
            return v

        def pix_body(g, c):
            base = lax.rem((j0 + g) * CHUNK, S)
            for gg in range(GRP):
                sl = pl.ds(g * CHUNK + gg * L, L)
                pv = base + gg * L + iota
                pv = jnp.where(pv >= S, pv - S, pv)
                pix_v[sl] = ttc_v[sl] * S + pv
            return c

        lax.fori_loop(0, per_w, pix_body, 0)

        def start_gathers(g, gbuf_t, pbuf_t, gsem_t, psem_t):
            sl = pl.ds(g * CHUNK, CHUNK)
            pltpu.async_copy(word_hbm.at[ids_v.at[sl]], gbuf_t, gsem_t)
            pltpu.async_copy(ptt_hbm.at[pix_v.at[sl]], pbuf_t, psem_t)

        def compute_chunk(gbuf, pbuf, obuf):
            def grp_body(gg, c):
                t0 = gg * L
                acc_s = jnp.zeros((L,), jnp.float32)
                acc_q = jnp.zeros((L,), jnp.float32)
                for u in range(L):
                    t = t0 + u
                    xs = []
                    for i in range(HV):
                        sl = pl.ds(i * L, L)
                        x = gbuf[t, sl] + pbuf[t, sl]
                        obuf[t, sl] = x
                        xs.append(x)
                    vs = list(xs)
                    qs = [x * x for x in xs]
                    while len(vs) > 1:
                        vs = [vs[i_] + vs[i_ + 1]
                              for i_ in range(0, len(vs), 2)]
                        qs = [qs[i_] + qs[i_ + 1]
                              for i_ in range(0, len(qs), 2)]
                    msk = iota == u
                    acc_s = jnp.where(msk, lane_sum(vs[0]), acc_s)
                    acc_q = jnp.where(msk, lane_sum(qs[0]), acc_q)
                mean16 = acc_s * inv_h
                var16 = acc_q * inv_h - mean16 * mean16
                r16 = _rsqrt_newton(var16 + EPS)
                for u in range(L):
                    t = t0 + u
                    ui = z16 + u
                    mb = _gather16(mean16, ui)
                    rb = _gather16(r16, ui)
                    for i in range(HV):
                        sl = pl.ds(i * L, L)
                        obuf[t, sl] = (obuf[t, sl] - mb) * rb * ws[i] + bs[i]
                return c

            lax.fori_loop(0, GRP, grp_body, 0)

        start_gathers(0, gbuf_a, pbuf_a, gsem_a, psem_a)

        def half(k_, g, gn, gbuf_t, pbuf_t, gbuf_n, pbuf_n, obuf_t,
                 gsem_t, psem_t, gsem_n, psem_n, ssem_t):
            start_gathers(gn, gbuf_n, pbuf_n, gsem_n, psem_n)
            sl = pl.ds(g * CHUNK, CHUNK)
            pltpu.make_async_copy(
                word_hbm.at[ids_v.at[sl]], gbuf_t, gsem_t).wait()
            pltpu.make_async_copy(
                ptt_hbm.at[pix_v.at[sl]], pbuf_t, psem_t).wait()

            @pl.when(k_ > 0)
            def _():
                pltpu.make_async_copy(
                    obuf_t, out_hbm.at[pl.ds((j0 + g) * CHUNK, CHUNK)],
                    ssem_t).wait()

            compute_chunk(gbuf_t, pbuf_t, obuf_t)
            pltpu.async_copy(
                obuf_t, out_hbm.at[pl.ds((j0 + g) * CHUNK, CHUNK)], ssem_t)

        def pair_body(k_, c):
            ga = 2 * k_
            gb = ga + 1
            gn = jnp.minimum(2 * k_ + 2, per_w - 1)
            half(k_, ga, gb, gbuf_a, pbuf_a, gbuf_b, pbuf_b, obuf_a,
                 gsem_a, psem_a, gsem_b, psem_b, ssem_a)
            half(k_, gb, gn, gbuf_b, pbuf_b, gbuf_a, pbuf_a, obuf_b,
                 gsem_b, psem_b, gsem_a, psem_a, ssem_b)
            return c

        lax.fori_loop(0, per_w // 2, pair_body, 0)

        pltpu.make_async_copy(
            word_hbm.at[ids_v.at[pl.ds(0, CHUNK)]], gbuf_a, gsem_a).wait()
        pltpu.make_async_copy(
            ptt_hbm.at[pix_v.at[pl.ds(0, CHUNK)]], pbuf_a, psem_a).wait()
        pltpu.make_async_copy(
            obuf_a, out_hbm.at[pl.ds(j0 * CHUNK, CHUNK)], ssem_a).wait()
        pltpu.make_async_copy(
            obuf_b, out_hbm.at[pl.ds(j0 * CHUNK, CHUNK)], ssem_b).wait()

    return k


def _build_ptt(pos_emb, type_emb, S, H):
    def body(pos_ref, type_ref, out_ref):
        p = pos_ref[pl.ds(0, S), :]
        out_ref[pl.ds(0, S), :] = p + type_ref[0:1, :]
        out_ref[pl.ds(S, S), :] = p + type_ref[1:2, :]

    return pl.pallas_call(
        body,
        out_shape=jax.ShapeDtypeStruct((2 * S, H), jnp.float32),
    )(pos_emb, type_emb)


def kernel(input_ids, token_type_ids, word_emb, pos_emb, type_emb,
           ln_weight, ln_bias):
    B, S = input_ids.shape
    V, H = word_emb.shape
    N = B * S
    ids2 = input_ids.astype(jnp.int32).reshape(N)
    tt2 = token_type_ids.astype(jnp.int32).reshape(N)
    ptt = _build_ptt(pos_emb, type_emb, S, H)
    k = _make_sc_kernel(B, S, H, V)
    out = k(ids2, tt2, word_emb, ptt, ln_weight, ln_bias)
    return out.reshape(B, S, H)


# speedup vs baseline: 10.9873x; 1.0164x over previous
import functools

import jax
import jax.numpy as jnp
from jax import lax
from jax.experimental import pallas as pl
from jax.experimental.pallas import tpu as pltpu
from jax.experimental.pallas import tpu_sc as plsc

NC, NS, L = 2, 16, 16
NW = NC * NS
CHUNK = 128
EPS = 1e-12


def _rsqrt_newton(a):
    ii = lax.bitcast_convert_type(a, jnp.int32)
    ii = jnp.full(ii.shape, 0x5F3759DF, jnp.int32) - (ii >> 1)
    y = lax.bitcast_convert_type(ii, jnp.float32)
    h = -0.5 * a
    for _ in range(2):
        y = y * (1.5 + h * y * y)
    return y


def _gather16(v, idx):
    return v.at[idx].get(mode="promise_in_bounds")


def _make_sc_kernel(B, S, H, V):
    N = B * S
    assert N % CHUNK == 0 and H == 128
    n_chunks = N // CHUNK
    assert n_chunks % NW == 0
    per_w = n_chunks // NW
    assert per_w % 2 == 0
    HV = H // L
    GRP = CHUNK // L
    inv_h = 1.0 / H

    mesh = plsc.VectorSubcoreMesh(core_axis_name="c", subcore_axis_name="s")

    @functools.partial(
        pl.kernel,
        out_type=jax.ShapeDtypeStruct((N, H), jnp.float32),
        mesh=mesh,
        compiler_params=pltpu.CompilerParams(needs_layout_passes=False),
        scratch_types=[
            pltpu.VMEM((H,), jnp.float32),
            pltpu.VMEM((H,), jnp.float32),
            pltpu.VMEM((per_w * CHUNK,), jnp.int32),
            pltpu.VMEM((per_w * CHUNK,), jnp.int32),
            pltpu.VMEM((per_w * CHUNK,), jnp.int32),
            pltpu.VMEM((CHUNK, H), jnp.float32),
            pltpu.VMEM((CHUNK, H), jnp.float32),
            pltpu.VMEM((CHUNK, H), jnp.float32),
            pltpu.VMEM((CHUNK, H), jnp.float32),
            pltpu.VMEM((CHUNK, H), jnp.float32),
            pltpu.VMEM((CHUNK, H), jnp.float32),
            pltpu.SemaphoreType.DMA,
            pltpu.SemaphoreType.DMA,
            pltpu.SemaphoreType.DMA,
            pltpu.SemaphoreType.DMA,
            pltpu.SemaphoreType.DMA,
            pltpu.SemaphoreType.DMA,
        ],
    )
    def k(ids_hbm, tt_hbm, word_hbm, ptt_hbm, w_hbm, b_hbm,
          out_hbm, w_v, b_v, ids_v, ttc_v, pix_v,
          gbuf_a, gbuf_b, pbuf_a, pbuf_b, obuf_a, obuf_b,
          gsem_a, gsem_b, psem_a, psem_b, ssem_a, ssem_b):
        wid = lax.axis_index("s") * NC + lax.axis_index("c")
        j0 = wid * per_w

        pltpu.sync_copy(w_hbm, w_v)
        pltpu.sync_copy(b_hbm, b_v)
        pltpu.sync_copy(ids_hbm.at[pl.ds(j0 * CHUNK, per_w * CHUNK)], ids_v)
        pltpu.sync_copy(tt_hbm.at[pl.ds(j0 * CHUNK, per_w * CHUNK)], ttc_v)

        ws = [w_v[pl.ds(i * L, L)] for i in range(HV)]
        bs = [b_v[pl.ds(i * L, L)] for i in range(HV)]

        iota = lax.iota(jnp.int32, L)
        z16 = iota * 0
        fifteen = z16 + (L - 1)

        def lane_sum(v):
            return _gather16(plsc.cumsum(v), fifteen)

        def pix_body(g, c):
            base = lax.rem((j0 + g) * CHUNK, S)
            for gg in range(GRP):
                sl = pl.ds(g * CHUNK + gg * L, L)
                pv = base + gg * L + iota
                pv = jnp.where(pv >= S, pv - S, pv)
                pix_v[sl] = ttc_v[sl] * S + pv
            return c

        lax.fori_loop(0, per_w, pix_body, 0)

        def start_gathers(g, gbuf_t, pbuf_t, gsem_t, psem_t):
            sl = pl.ds(g * CHUNK, CHUNK)
            pltpu.async_copy(word_hbm.at[ids_v.at[sl]], gbuf_t, gsem_t)
            pltpu.async_copy(ptt_hbm.at[pix_v.at[sl]], pbuf_t, psem_t)

        def compute_chunk(gbuf, pbuf, obuf):
            def grp_body(gg, c):
                t0 = gg * L
                acc_s = jnp.zeros((L,), jnp.float32)
                acc_q = jnp.zeros((L,), jnp.float32)
                for u in range(L):
                    t = t0 + u
                    xs = []
                    for i in range(HV):
                        sl = pl.ds(i * L, L)
                        x = gbuf[t, sl] + pbuf[t, sl]
                        obuf[t, sl] = x
                        xs.append(x)
                    vs = list(xs)
                    qs = [x * x for x in xs]
                    while len(vs) > 1:
                        vs = [vs[i_] + vs[i_ + 1]
                              for i_ in range(0, len(vs), 2)]
                        qs = [qs[i_] + qs[i_ + 1]
                              for i_ in range(0, len(qs), 2)]
                    msk = iota == u
                    acc_s = jnp.where(msk, lane_sum(vs[0]), acc_s)
                    acc_q = jnp.where(msk, lane_sum(qs[0]), acc_q)
                mean16 = acc_s * inv_h
                var16 = acc_q * inv_h - mean16 * mean16
                r16 = _rsqrt_newton(var16 + EPS)
                for u in range(L):
                    t = t0 + u
                    ui = z16 + u
                    mb = _gather16(mean16, ui)
                    rb = _gather16(r16, ui)
                    for i in range(HV):
                        sl = pl.ds(i * L, L)
                        obuf[t, sl] = (obuf[t, sl] - mb) * rb * ws[i] + bs[i]
                return c

            lax.fori_loop(0, GRP, grp_body, 0)

        start_gathers(0, gbuf_a, pbuf_a, gsem_a, psem_a)

        def half(k_, g, gn, gbuf_t, pbuf_t, gbuf_n, pbuf_n, obuf_t,
                 gsem_t, psem_t, gsem_n, psem_n, ssem_t):
            start_gathers(gn, gbuf_n, pbuf_n, gsem_n, psem_n)
            sl = pl.ds(g * CHUNK, CHUNK)
            pltpu.make_async_copy(
                word_hbm.at[ids_v.at[sl]], gbuf_t, gsem_t).wait()
            pltpu.make_async_copy(
                ptt_hbm.at[pix_v.at[sl]], pbuf_t, psem_t).wait()

            @pl.when(k_ > 0)
            def _():
                pltpu.make_async_copy(
                    obuf_t, out_hbm.at[pl.ds((j0 + g) * CHUNK, CHUNK)],
                    ssem_t).wait()

            compute_chunk(gbuf_t, pbuf_t, obuf_t)
            pltpu.async_copy(
                obuf_t, out_hbm.at[pl.ds((j0 + g) * CHUNK, CHUNK)], ssem_t)

        def pair_body(k_, c):
            ga = 2 * k_
            gb = ga + 1
            gn = jnp.minimum(2 * k_ + 2, per_w - 1)
            half(k_, ga, gb, gbuf_a, pbuf_a, gbuf_b, pbuf_b, obuf_a,
                 gsem_a, psem_a, gsem_b, psem_b, ssem_a)
            half(k_, gb, gn, gbuf_b, pbuf_b, gbuf_a, pbuf_a, obuf_b,
                 gsem_b, psem_b, gsem_a, psem_a, ssem_b)
            return c

        lax.fori_loop(0, per_w // 2, pair_body, 0)

        pltpu.make_async_copy(
            word_hbm.at[ids_v.at[pl.ds(0, CHUNK)]], gbuf_a, gsem_a).wait()
        pltpu.make_async_copy(
            ptt_hbm.at[pix_v.at[pl.ds(0, CHUNK)]], pbuf_a, psem_a).wait()
        pltpu.make_async_copy(
            obuf_a, out_hbm.at[pl.ds(j0 * CHUNK, CHUNK)], ssem_a).wait()
        pltpu.make_async_copy(
            obuf_b, out_hbm.at[pl.ds(j0 * CHUNK, CHUNK)], ssem_b).wait()

    return k


def _build_ptt(pos_emb, type_emb, S, H):
    def body(pos_ref, type_ref, out_ref):
        p = pos_ref[pl.ds(0, S), :]
        out_ref[pl.ds(0, S), :] = p + type_ref[0:1, :]
        out_ref[pl.ds(S, S), :] = p + type_ref[1:2, :]

    return pl.pallas_call(
        body,
        out_shape=jax.ShapeDtypeStruct((2 * S, H), jnp.float32),
    )(pos_emb, type_emb)


def kernel(input_ids, token_type_ids, word_emb, pos_emb, type_emb,
           ln_weight, ln_bias):
    B, S = input_ids.shape
    V, H = word_emb.shape
    N = B * S
    ids2 = input_ids.astype(jnp.int32).reshape(N)
    tt2 = token_type_ids.astype(jnp.int32).reshape(N)
    ptt = _build_ptt(pos_emb, type_emb, S, H)
    k = _make_sc_kernel(B, S, H, V)
    out = k(ids2, tt2, word_emb, ptt, ln_weight, ln_bias)
    return out.reshape(B, S, H)


# speedup vs baseline: 11.1989x; 1.0193x over previous
import functools

import jax
import jax.numpy as jnp
from jax import lax
from jax.experimental import pallas as pl
from jax.experimental.pallas import tpu as pltpu
from jax.experimental.pallas import tpu_sc as plsc

NC, NS, L = 2, 16, 16
NW = NC * NS
CHUNK = 128
EPS = 1e-12


def _rsqrt_newton(a):
    ii = lax.bitcast_convert_type(a, jnp.int32)
    ii = jnp.full(ii.shape, 0x5F3759DF, jnp.int32) - (ii >> 1)
    y = lax.bitcast_convert_type(ii, jnp.float32)
    h = -0.5 * a
    for _ in range(2):
        y = y * (1.5 + h * y * y)
    return y


def _gather16(v, idx):
    return v.at[idx].get(mode="promise_in_bounds")


def _make_sc_kernel(B, S, H, V):
    N = B * S
    assert N % CHUNK == 0 and H == 128
    n_chunks = N // CHUNK
    assert n_chunks % NW == 0
    per_w = n_chunks // NW
    assert per_w % 2 == 0
    HV = H // L
    GRP = CHUNK // L
    inv_h = 1.0 / H

    mesh = plsc.VectorSubcoreMesh(core_axis_name="c", subcore_axis_name="s")

    @functools.partial(
        pl.kernel,
        out_type=jax.ShapeDtypeStruct((N, H), jnp.float32),
        mesh=mesh,
        compiler_params=pltpu.CompilerParams(needs_layout_passes=False),
        scratch_types=[
            pltpu.VMEM((H,), jnp.float32),
            pltpu.VMEM((H,), jnp.float32),
            pltpu.VMEM((per_w * CHUNK,), jnp.int32),
            pltpu.VMEM((per_w * CHUNK,), jnp.int32),
            pltpu.VMEM((per_w * CHUNK,), jnp.int32),
            pltpu.VMEM((CHUNK, H), jnp.float32),
            pltpu.VMEM((CHUNK, H), jnp.float32),
            pltpu.VMEM((CHUNK, H), jnp.float32),
            pltpu.VMEM((CHUNK, H), jnp.float32),
            pltpu.VMEM((CHUNK, H), jnp.float32),
            pltpu.VMEM((CHUNK, H), jnp.float32),
            pltpu.VMEM((CHUNK // L, L), jnp.float32),
            pltpu.VMEM((CHUNK // L, L), jnp.float32),
            pltpu.SemaphoreType.DMA,
            pltpu.SemaphoreType.DMA,
            pltpu.SemaphoreType.DMA,
            pltpu.SemaphoreType.DMA,
            pltpu.SemaphoreType.DMA,
            pltpu.SemaphoreType.DMA,
        ],
    )
    def k(ids_hbm, tt_hbm, word_hbm, ptt_hbm, w_hbm, b_hbm,
          out_hbm, w_v, b_v, ids_v, ttc_v, pix_v,
          gbuf_a, gbuf_b, pbuf_a, pbuf_b, obuf_a, obuf_b, mv_v, rv_v,
          gsem_a, gsem_b, psem_a, psem_b, ssem_a, ssem_b):
        wid = lax.axis_index("s") * NC + lax.axis_index("c")
        j0 = wid * per_w

        pltpu.sync_copy(w_hbm, w_v)
        pltpu.sync_copy(b_hbm, b_v)
        pltpu.sync_copy(ids_hbm.at[pl.ds(j0 * CHUNK, per_w * CHUNK)], ids_v)
        pltpu.sync_copy(tt_hbm.at[pl.ds(j0 * CHUNK, per_w * CHUNK)], ttc_v)

        iota = lax.iota(jnp.int32, L)
        z16 = iota * 0
        fifteen = z16 + (L - 1)

        def lane_sum(v):
            return _gather16(plsc.cumsum(v), fifteen)

        def pix_body(g, c):
            base = lax.rem((j0 + g) * CHUNK, S)
            for gg in range(GRP):
                sl = pl.ds(g * CHUNK + gg * L, L)
                pv = base + gg * L + iota
                pv = jnp.where(pv >= S, pv - S, pv)
                pix_v[sl] = ttc_v[sl] * S + pv
            return c

        lax.fori_loop(0, per_w, pix_body, 0)

        def start_gathers(g, gbuf_t, pbuf_t, gsem_t, psem_t):
            sl = pl.ds(g * CHUNK, CHUNK)
            pltpu.async_copy(word_hbm.at[ids_v.at[sl]], gbuf_t, gsem_t)
            pltpu.async_copy(ptt_hbm.at[pix_v.at[sl]], pbuf_t, psem_t)

        def compute_chunk(gbuf, pbuf, obuf):
            def p1_body(gg, c):
                t0 = gg * L
                acc_s = jnp.zeros((L,), jnp.float32)
                acc_q = jnp.zeros((L,), jnp.float32)
                for u in range(L):
                    t = t0 + u
                    xs = []
                    for i in range(HV):
                        sl = pl.ds(i * L, L)
                        x = gbuf[t, sl] + pbuf[t, sl]
                        obuf[t, sl] = x
                        xs.append(x)
                    vs = list(xs)
                    qs = [x * x for x in xs]
                    while len(vs) > 1:
                        vs = [vs[i_] + vs[i_ + 1]
                              for i_ in range(0, len(vs), 2)]
                        qs = [qs[i_] + qs[i_ + 1]
                              for i_ in range(0, len(qs), 2)]
                    msk = iota == u
                    acc_s = jnp.where(msk, lane_sum(vs[0]), acc_s)
                    acc_q = jnp.where(msk, lane_sum(qs[0]), acc_q)
                mean16 = acc_s * inv_h
                var16 = acc_q * inv_h - mean16 * mean16
                mv_v[gg, :] = mean16
                rv_v[gg, :] = _rsqrt_newton(var16 + EPS)
                return c

            lax.fori_loop(0, GRP, p1_body, 0)

            def p2_body(gg, c):
                t0 = gg * L
                mean16 = mv_v[gg, :]
                r16 = rv_v[gg, :]
                ws = [w_v[pl.ds(i * L, L)] for i in range(HV)]
                bs = [b_v[pl.ds(i * L, L)] for i in range(HV)]
                for u in range(L):
                    t = t0 + u
                    ui = z16 + u
                    mb = _gather16(mean16, ui)
                    rb = _gather16(r16, ui)
                    for i in range(HV):
                        sl = pl.ds(i * L, L)
                        obuf[t, sl] = (obuf[t, sl] - mb) * rb * ws[i] + bs[i]
                return c

            lax.fori_loop(0, GRP, p2_body, 0)

        start_gathers(0, gbuf_a, pbuf_a, gsem_a, psem_a)

        def half(k_, g, gn, gbuf_t, pbuf_t, gbuf_n, pbuf_n, obuf_t,
                 gsem_t, psem_t, gsem_n, psem_n, ssem_t):
            start_gathers(gn, gbuf_n, pbuf_n, gsem_n, psem_n)
            sl = pl.ds(g * CHUNK, CHUNK)
            pltpu.make_async_copy(
                word_hbm.at[ids_v.at[sl]], gbuf_t, gsem_t).wait()
            pltpu.make_async_copy(
                ptt_hbm.at[pix_v.at[sl]], pbuf_t, psem_t).wait()

            @pl.when(k_ > 0)
            def _():
                pltpu.make_async_copy(
                    obuf_t, out_hbm.at[pl.ds((j0 + g) * CHUNK, CHUNK)],
                    ssem_t).wait()

            compute_chunk(gbuf_t, pbuf_t, obuf_t)
            pltpu.async_copy(
                obuf_t, out_hbm.at[pl.ds((j0 + g) * CHUNK, CHUNK)], ssem_t)

        def pair_body(k_, c):
            ga = 2 * k_
            gb = ga + 1
            gn = jnp.minimum(2 * k_ + 2, per_w - 1)
            half(k_, ga, gb, gbuf_a, pbuf_a, gbuf_b, pbuf_b, obuf_a,
                 gsem_a, psem_a, gsem_b, psem_b, ssem_a)
            half(k_, gb, gn, gbuf_b, pbuf_b, gbuf_a, pbuf_a, obuf_b,
                 gsem_b, psem_b, gsem_a, psem_a, ssem_b)
            return c

        lax.fori_loop(0, per_w // 2, pair_body, 0)

        pltpu.make_async_copy(
            word_hbm.at[ids_v.at[pl.ds(0, CHUNK)]], gbuf_a, gsem_a).wait()
        pltpu.make_async_copy(
            ptt_hbm.at[pix_v.at[pl.ds(0, CHUNK)]], pbuf_a, psem_a).wait()
        pltpu.make_async_copy(
            obuf_a, out_hbm.at[pl.ds(j0 * CHUNK, CHUNK)], ssem_a).wait()
        pltpu.make_async_copy(
            obuf_b, out_hbm.at[pl.ds(j0 * CHUNK, CHUNK)], ssem_b).wait()

    return k


def _build_ptt(pos_emb, type_emb, S, H):
    def body(pos_ref, type_ref, out_ref):
        p = pos_ref[pl.ds(0, S), :]
        out_ref[pl.ds(0, S), :] = p + type_ref[0:1, :]
        out_ref[pl.ds(S, S), :] = p + type_ref[1:2, :]

    return pl.pallas_call(
        body,
        out_shape=jax.ShapeDtypeStruct((2 * S, H), jnp.float32),
    )(pos_emb, type_emb)


def kernel(input_ids, token_type_ids, word_emb, pos_emb, type_emb,
           ln_weight, ln_bias):
    B, S = input_ids.shape
    V, H = word_emb.shape
    N = B * S
    ids2 = input_ids.astype(jnp.int32).reshape(N)
    tt2 = token_type_ids.astype(jnp.int32).reshape(N)
    ptt = _build_ptt(pos_emb, type_emb, S, H)
    k = _make_sc_kernel(B, S, H, V)
    out = k(ids2, tt2, word_emb, ptt, ln_weight, ln_bias)
    return out.reshape(B, S, H)
